# baseline (speedup 1.0000x reference)
import functools

import jax
import jax.numpy as jnp
from jax import lax
from jax.experimental import pallas as pl
from jax.experimental.pallas import tpu as pltpu
from jax.experimental.pallas import tpu_sc as plsc

NC = 2
NS = 16
NW = NC * NS

NEMB = 1_000_000
DIM = 16
BATCH = 16384
NF = 26
NUM_ROWS = BATCH * NF

TBLK = 512
NBLK_FULL = NEMB // TBLK
TAIL = NEMB - NBLK_FULL * TBLK
BLK_PER_W = NBLK_FULL // NW

BSTRIPE = BATCH // NW

CHUNK = 1024
NCH_W = NUM_ROWS // NW // CHUNK
CH_PER_F = BATCH // CHUNK
JSPAN = 8 * 128 * 8 + 7 * 128 + 8


@functools.partial(
    pl.kernel,
    mesh=plsc.VectorSubcoreMesh(core_axis_name="c", subcore_axis_name="s"),
    out_type=(
        jax.ShapeDtypeStruct((NEMB * DIM,), jnp.float32),
        jax.ShapeDtypeStruct((NUM_ROWS,), jnp.int32),
    ),
    scratch_types=[
        pltpu.VMEM((16, TBLK), jnp.float32),
        pltpu.VMEM((16, TBLK), jnp.float32),
        pltpu.VMEM((16, TBLK), jnp.float32),
        pltpu.VMEM((TBLK * DIM,), jnp.float32),
        pltpu.VMEM((TBLK * DIM,), jnp.float32),
        pltpu.VMEM((TBLK * DIM,), jnp.float32),
        pltpu.VMEM((NF, BSTRIPE), jnp.int32),
        pltpu.VMEM((NF * BSTRIPE,), jnp.int32),
        pltpu.VMEM((TAIL, DIM), jnp.float32),
        pltpu.SemaphoreType.DMA,
        pltpu.SemaphoreType.DMA,
        pltpu.SemaphoreType.DMA,
        pltpu.SemaphoreType.DMA,
        pltpu.SemaphoreType.DMA,
        pltpu.SemaphoreType.DMA,
    ],
    compiler_params=pltpu.CompilerParams(use_tc_tiling_on_sc=True,
                                         needs_layout_passes=False),
)
def _stage1(wt_hbm, it_hbm, tail_hbm, tbl_hbm, idx_hbm,
            src0, src1, src2, stg0, stg1, stg2, itile, ilin, tail_v,
            gsem0, gsem1, gsem2, osem0, osem1, osem2):
    wid = lax.axis_index("s") * NC + lax.axis_index("c")
    lanes16 = lax.iota(jnp.int32, 16)
    ivec = lanes16 * DIM

    pltpu.sync_copy(
        it_hbm.at[:, pl.ds(pl.multiple_of(wid * BSTRIPE, BSTRIPE), BSTRIPE)],
        itile)

    def detile_body(i, _):
        f = i // (BSTRIPE // 16)
        c = i % (BSTRIPE // 16)
        ilin[pl.ds(pl.multiple_of(i * 16, 16), 16)] = (
            itile[f, pl.ds(pl.multiple_of(c * 16, 16), 16)])
        return 0
    lax.fori_loop(0, NF * (BSTRIPE // 16), detile_body, 0)
    for f in range(NF):
        pltpu.sync_copy(
            ilin.at[pl.ds(f * BSTRIPE, BSTRIPE)],
            idx_hbm.at[pl.ds(pl.multiple_of(f * BATCH + wid * BSTRIPE, BSTRIPE),
                             BSTRIPE)])

    srcs = (src0, src1, src2)
    stgs = (stg0, stg1, stg2)
    gsems = (gsem0, gsem1, gsem2)
    osems = (osem0, osem1, osem2)

    def blk_lane(k):
        return pl.multiple_of((k * NW + wid) * TBLK, TBLK)

    def blk_word(k):
        return pl.multiple_of((k * NW + wid) * TBLK * DIM, TBLK * DIM)

    def gwait(b):
        pltpu.make_async_copy(
            wt_hbm.at[:, pl.ds(0, TBLK)], srcs[b], gsems[b]).wait()

    def owait(b):
        pltpu.make_async_copy(
            stgs[b], tbl_hbm.at[pl.ds(0, TBLK * DIM)], osems[b]).wait()

    jvecs = [ivec + j for j in range(16)]

    def transpose(src, stg):
        def body(t, _):
            base = pl.multiple_of(t * 128, 128)
            woff = pl.multiple_of(t * (128 * DIM), 128 * DIM)
            for q in range(8):
                window = stg.at[pl.ds(woff + q * (16 * DIM), 16 * DIM)]
                for j in range(16):
                    v = src[j, pl.ds(pl.multiple_of(base + q * 16, 16), 16)]
                    plsc.store_scatter(window, [jvecs[j]], v)
            return 0
        lax.fori_loop(0, TBLK // 128, body, 0)

    pltpu.async_copy(wt_hbm.at[:, pl.ds(blk_lane(0), TBLK)], src0, gsem0)
    pltpu.async_copy(wt_hbm.at[:, pl.ds(blk_lane(1), TBLK)], src1, gsem1)
    pltpu.async_copy(wt_hbm.at[:, pl.ds(blk_lane(2), TBLK)], src2, gsem2)

    def triple_body(i, _):
        for b in range(3):
            k = 3 * i + b
            gwait(b)

            @pl.when(i > 0)
            def _():
                owait(b)

            transpose(srcs[b], stgs[b])
            pltpu.async_copy(
                stgs[b], tbl_hbm.at[pl.ds(blk_word(k), TBLK * DIM)], osems[b])

            @pl.when(k + 3 < BLK_PER_W)
            def _():
                pltpu.async_copy(
                    wt_hbm.at[:, pl.ds(blk_lane(k + 3), TBLK)],
                    srcs[b], gsems[b])
        return 0

    lax.fori_loop(0, (BLK_PER_W - 1) // 3, triple_body, 0)
    gwait(0)
    owait(0)
    transpose(src0, stg0)
    pltpu.sync_copy(stg0, tbl_hbm.at[pl.ds(blk_word(BLK_PER_W - 1), TBLK * DIM)])
    owait(1)
    owait(2)

    @pl.when(wid == 0)
    def _():
        lane0 = (NBLK_FULL - 1) * TBLK
        pltpu.sync_copy(wt_hbm.at[:, pl.ds(lane0, TBLK)], src0)
        transpose(src0, stg0)
        pltpu.sync_copy(stg0, tbl_hbm.at[pl.ds(lane0 * DIM, TBLK * DIM)])

        pltpu.sync_copy(tail_hbm, tail_v)
        for r in range(TAIL):
            stg1[pl.ds(r * DIM, 16)] = tail_v[r, pl.ds(0, 16)]
        pltpu.sync_copy(stg1.at[pl.ds(0, TAIL * DIM)],
                        tbl_hbm.at[pl.ds(NBLK_FULL * TBLK * DIM, TAIL * DIM)])


@functools.partial(
    pl.kernel,
    mesh=plsc.VectorSubcoreMesh(core_axis_name="c", subcore_axis_name="s"),
    out_type=jax.ShapeDtypeStruct((NF * 2 * (BATCH // 128) * 8 * 128,), jnp.float32),
    scratch_types=[
        pltpu.VMEM((NCH_W * CHUNK,), jnp.int32),
        pltpu.VMEM((CHUNK, DIM), jnp.float32),
        pltpu.VMEM((CHUNK, DIM), jnp.float32),
        pltpu.VMEM((CHUNK, DIM), jnp.float32),
        pltpu.VMEM((CHUNK * DIM,), jnp.float32),
        pltpu.VMEM((CHUNK * DIM,), jnp.float32),
        pltpu.SemaphoreType.DMA,
        pltpu.SemaphoreType.DMA,
        pltpu.SemaphoreType.DMA,
        pltpu.SemaphoreType.DMA,
        pltpu.SemaphoreType.DMA,
    ],
    compiler_params=pltpu.CompilerParams(use_tc_tiling_on_sc=False,
                                         needs_layout_passes=False),
)
def _stage2(tbl_hbm, idx_hbm, out_hbm,
            idx_v, rows0, rows1, rows2, stg0, stg1,
            gsem0, gsem1, gsem2, osem0, osem1):
    wid = lax.axis_index("s") * NC + lax.axis_index("c")
    base = pl.multiple_of(wid * NCH_W * CHUNK, CHUNK)
    lanes16 = lax.iota(jnp.int32, 16)
    jmap = (lanes16 // 8) * (8 * 128 * 8) + (lanes16 % 8) * 128
    jmaps = [jmap + u for u in range(8)]
    pltpu.sync_copy(idx_hbm.at[pl.ds(base, NCH_W * CHUNK)], idx_v)

    rows = (rows0, rows1, rows2)
    stgs = (stg0, stg1)
    gsems = (gsem0, gsem1, gsem2)
    osems = (osem0, osem1)
    hands = [None, None, None]
    owaits = [None, None]

    hands[0] = pltpu.async_copy(
        tbl_hbm.at[idx_v.at[pl.ds(0, CHUNK)]], rows0, gsem0)
    hands[1] = pltpu.async_copy(
        tbl_hbm.at[idx_v.at[pl.ds(CHUNK, CHUNK)]], rows1, gsem1)

    cg0 = wid * NCH_W
    for c in range(NCH_W):
        s3 = c % 3
        s2 = c % 2
        if c + 2 < NCH_W:
            hands[(c + 2) % 3] = pltpu.async_copy(
                tbl_hbm.at[idx_v.at[pl.ds((c + 2) * CHUNK, CHUNK)]],
                rows[(c + 2) % 3], gsems[(c + 2) % 3])
        hands[s3].wait()
        if owaits[s2] is not None:
            for ow in owaits[s2]:
                ow.wait()
            owaits[s2] = None

        def body(i, _):
            pos = (i // 16) * (8 * 128) + (i % 16) * 8
            window = stgs[s2].at[pl.ds(pl.multiple_of(pos, 8), JSPAN)]
            for u in range(8):
                b = i * 8 + u
                v = rows[s3][b, pl.ds(0, 16)]
                plsc.store_scatter(window, [jmaps[u]], v)
            return 0
        lax.fori_loop(0, CHUNK // 8, body, 0)

        cg = cg0 + c
        f = cg // CH_PER_F
        bc0 = (cg % CH_PER_F) * 8
        half = 8 * 128 * 8
        off0 = pl.multiple_of(((f * 2 + 0) * (BATCH // 128) + bc0) * 1024, 1024)
        off1 = pl.multiple_of(((f * 2 + 1) * (BATCH // 128) + bc0) * 1024, 1024)
        ow0 = pltpu.async_copy(
            stgs[s2].at[pl.ds(0, half)], out_hbm.at[pl.ds(off0, half)], osems[s2])
        ow1 = pltpu.async_copy(
            stgs[s2].at[pl.ds(half, half)], out_hbm.at[pl.ds(off1, half)], osems[s2])
        owaits[s2] = (ow0, ow1)

    for s in range(2):
        if owaits[s] is not None:
            for ow in owaits[s]:
                ow.wait()


def kernel(input, weight):
    wt = weight.T
    it = input.T
    tail = lax.slice(weight, (NEMB - TAIL, 0), (NEMB, DIM))
    tbl_flat, idx_lin = _stage1(wt, it, tail)
    table = tbl_flat.reshape(NEMB, DIM)
    out_flat = _stage2(table, idx_lin)
    out5 = out_flat.reshape(NF, 2, BATCH // 128, 8, 128)
    return out5.transpose(2, 4, 0, 1, 3).reshape(BATCH, NF, DIM)


# speedup vs baseline: 1.0905x; 1.0905x over previous
import functools

import jax
import jax.numpy as jnp
from jax import lax
from jax.experimental import pallas as pl
from jax.experimental.pallas import tpu as pltpu
from jax.experimental.pallas import tpu_sc as plsc

NC = 2
NS = 16
NW = NC * NS

NEMB = 1_000_000
DIM = 16
BATCH = 16384
NF = 26
NUM_ROWS = BATCH * NF

TBLK = 512
NBLK_FULL = NEMB // TBLK
TAIL = NEMB - NBLK_FULL * TBLK
BLK_PER_W = NBLK_FULL // NW

BSTRIPE = BATCH // NW

CHUNK = 1024
NCH_W = NUM_ROWS // NW // CHUNK
CH_PER_F = BATCH // CHUNK
JSPAN = 8 * 128 * 8 + 7 * 128 + 8


@functools.partial(
    pl.kernel,
    mesh=plsc.VectorSubcoreMesh(core_axis_name="c", subcore_axis_name="s"),
    out_type=(
        jax.ShapeDtypeStruct((NEMB * DIM,), jnp.float32),
        jax.ShapeDtypeStruct((NUM_ROWS,), jnp.int32),
    ),
    scratch_types=[
        pltpu.VMEM((16, TBLK), jnp.float32),
        pltpu.VMEM((16, TBLK), jnp.float32),
        pltpu.VMEM((16, TBLK), jnp.float32),
        pltpu.VMEM((TBLK * DIM,), jnp.float32),
        pltpu.VMEM((TBLK * DIM,), jnp.float32),
        pltpu.VMEM((TBLK * DIM,), jnp.float32),
        pltpu.VMEM((NF, BSTRIPE), jnp.int32),
        pltpu.VMEM((NF * BSTRIPE,), jnp.int32),
        pltpu.VMEM((TAIL, DIM), jnp.float32),
        pltpu.SemaphoreType.DMA,
        pltpu.SemaphoreType.DMA,
        pltpu.SemaphoreType.DMA,
        pltpu.SemaphoreType.DMA,
        pltpu.SemaphoreType.DMA,
        pltpu.SemaphoreType.DMA,
    ],
    compiler_params=pltpu.CompilerParams(use_tc_tiling_on_sc=True,
                                         needs_layout_passes=False),
)
def _stage1(wt_hbm, it_hbm, tail_hbm, tbl_hbm, idx_hbm,
            src0, src1, src2, stg0, stg1, stg2, itile, ilin, tail_v,
            gsem0, gsem1, gsem2, osem0, osem1, osem2):
    wid = lax.axis_index("s") * NC + lax.axis_index("c")
    lanes16 = lax.iota(jnp.int32, 16)
    ivec = lanes16 * DIM

    pltpu.sync_copy(
        it_hbm.at[:, pl.ds(pl.multiple_of(wid * BSTRIPE, BSTRIPE), BSTRIPE)],
        itile)

    @plsc.parallel_loop(0, NF * (BSTRIPE // 16))
    def detile_body(i):
        f = i // (BSTRIPE // 16)
        c = i % (BSTRIPE // 16)
        ilin[pl.ds(pl.multiple_of(i * 16, 16), 16)] = (
            itile[f, pl.ds(pl.multiple_of(c * 16, 16), 16)])
    for f in range(NF):
        pltpu.sync_copy(
            ilin.at[pl.ds(f * BSTRIPE, BSTRIPE)],
            idx_hbm.at[pl.ds(pl.multiple_of(f * BATCH + wid * BSTRIPE, BSTRIPE),
                             BSTRIPE)])

    srcs = (src0, src1, src2)
    stgs = (stg0, stg1, stg2)
    gsems = (gsem0, gsem1, gsem2)
    osems = (osem0, osem1, osem2)

    def blk_lane(k):
        return pl.multiple_of((k * NW + wid) * TBLK, TBLK)

    def blk_word(k):
        return pl.multiple_of((k * NW + wid) * TBLK * DIM, TBLK * DIM)

    def gwait(b):
        pltpu.make_async_copy(
            wt_hbm.at[:, pl.ds(0, TBLK)], srcs[b], gsems[b]).wait()

    def owait(b):
        pltpu.make_async_copy(
            stgs[b], tbl_hbm.at[pl.ds(0, TBLK * DIM)], osems[b]).wait()

    jvecs = [ivec + j for j in range(16)]

    def transpose(src, stg):
        @plsc.parallel_loop(0, TBLK // 128)
        def body(t):
            base = pl.multiple_of(t * 128, 128)
            woff = pl.multiple_of(t * (128 * DIM), 128 * DIM)
            for q in range(8):
                window = stg.at[pl.ds(woff + q * (16 * DIM), 16 * DIM)]
                for j in range(16):
                    v = src[j, pl.ds(pl.multiple_of(base + q * 16, 16), 16)]
                    plsc.store_scatter(window, [jvecs[j]], v)

    pltpu.async_copy(wt_hbm.at[:, pl.ds(blk_lane(0), TBLK)], src0, gsem0)
    pltpu.async_copy(wt_hbm.at[:, pl.ds(blk_lane(1), TBLK)], src1, gsem1)
    pltpu.async_copy(wt_hbm.at[:, pl.ds(blk_lane(2), TBLK)], src2, gsem2)

    def triple_body(i, _):
        for b in range(3):
            k = 3 * i + b
            gwait(b)

            @pl.when(i > 0)
            def _():
                owait(b)

            transpose(srcs[b], stgs[b])
            pltpu.async_copy(
                stgs[b], tbl_hbm.at[pl.ds(blk_word(k), TBLK * DIM)], osems[b])

            @pl.when(k + 3 < BLK_PER_W)
            def _():
                pltpu.async_copy(
                    wt_hbm.at[:, pl.ds(blk_lane(k + 3), TBLK)],
                    srcs[b], gsems[b])
        return 0

    lax.fori_loop(0, (BLK_PER_W - 1) // 3, triple_body, 0)
    gwait(0)
    owait(0)
    transpose(src0, stg0)
    pltpu.sync_copy(stg0, tbl_hbm.at[pl.ds(blk_word(BLK_PER_W - 1), TBLK * DIM)])
    owait(1)
    owait(2)

    @pl.when(wid == 0)
    def _():
        lane0 = (NBLK_FULL - 1) * TBLK
        pltpu.sync_copy(wt_hbm.at[:, pl.ds(lane0, TBLK)], src0)
        transpose(src0, stg0)
        pltpu.sync_copy(stg0, tbl_hbm.at[pl.ds(lane0 * DIM, TBLK * DIM)])

        pltpu.sync_copy(tail_hbm, tail_v)
        for r in range(TAIL):
            stg1[pl.ds(r * DIM, 16)] = tail_v[r, pl.ds(0, 16)]
        pltpu.sync_copy(stg1.at[pl.ds(0, TAIL * DIM)],
                        tbl_hbm.at[pl.ds(NBLK_FULL * TBLK * DIM, TAIL * DIM)])


@functools.partial(
    pl.kernel,
    mesh=plsc.VectorSubcoreMesh(core_axis_name="c", subcore_axis_name="s"),
    out_type=jax.ShapeDtypeStruct((NF * 2 * (BATCH // 128) * 8 * 128,), jnp.float32),
    scratch_types=[
        pltpu.VMEM((NCH_W * CHUNK,), jnp.int32),
        pltpu.VMEM((CHUNK, DIM), jnp.float32),
        pltpu.VMEM((CHUNK, DIM), jnp.float32),
        pltpu.VMEM((CHUNK, DIM), jnp.float32),
        pltpu.VMEM((CHUNK * DIM,), jnp.float32),
        pltpu.VMEM((CHUNK * DIM,), jnp.float32),
        pltpu.SemaphoreType.DMA,
        pltpu.SemaphoreType.DMA,
        pltpu.SemaphoreType.DMA,
        pltpu.SemaphoreType.DMA,
        pltpu.SemaphoreType.DMA,
    ],
    compiler_params=pltpu.CompilerParams(use_tc_tiling_on_sc=False,
                                         needs_layout_passes=False),
)
def _stage2(tbl_hbm, idx_hbm, out_hbm,
            idx_v, rows0, rows1, rows2, stg0, stg1,
            gsem0, gsem1, gsem2, osem0, osem1):
    wid = lax.axis_index("s") * NC + lax.axis_index("c")
    base = pl.multiple_of(wid * NCH_W * CHUNK, CHUNK)
    lanes16 = lax.iota(jnp.int32, 16)
    jmap = (lanes16 // 8) * (8 * 128 * 8) + (lanes16 % 8) * 128
    jmaps = [jmap + u for u in range(8)]
    pltpu.sync_copy(idx_hbm.at[pl.ds(base, NCH_W * CHUNK)], idx_v)

    rows = (rows0, rows1, rows2)
    stgs = (stg0, stg1)
    gsems = (gsem0, gsem1, gsem2)
    osems = (osem0, osem1)
    hands = [None, None, None]
    owaits = [None, None]

    hands[0] = pltpu.async_copy(
        tbl_hbm.at[idx_v.at[pl.ds(0, CHUNK)]], rows0, gsem0)
    hands[1] = pltpu.async_copy(
        tbl_hbm.at[idx_v.at[pl.ds(CHUNK, CHUNK)]], rows1, gsem1)

    cg0 = wid * NCH_W
    for c in range(NCH_W):
        s3 = c % 3
        s2 = c % 2
        if c + 2 < NCH_W:
            hands[(c + 2) % 3] = pltpu.async_copy(
                tbl_hbm.at[idx_v.at[pl.ds((c + 2) * CHUNK, CHUNK)]],
                rows[(c + 2) % 3], gsems[(c + 2) % 3])
        hands[s3].wait()
        if owaits[s2] is not None:
            for ow in owaits[s2]:
                ow.wait()
            owaits[s2] = None

        @plsc.parallel_loop(0, CHUNK // 8)
        def body(i):
            pos = (i // 16) * (8 * 128) + (i % 16) * 8
            window = stgs[s2].at[pl.ds(pl.multiple_of(pos, 8), JSPAN)]
            for u in range(8):
                b = i * 8 + u
                v = rows[s3][b, pl.ds(0, 16)]
                plsc.store_scatter(window, [jmaps[u]], v)

        cg = cg0 + c
        f = cg // CH_PER_F
        bc0 = (cg % CH_PER_F) * 8
        half = 8 * 128 * 8
        off0 = pl.multiple_of(((f * 2 + 0) * (BATCH // 128) + bc0) * 1024, 1024)
        off1 = pl.multiple_of(((f * 2 + 1) * (BATCH // 128) + bc0) * 1024, 1024)
        ow0 = pltpu.async_copy(
            stgs[s2].at[pl.ds(0, half)], out_hbm.at[pl.ds(off0, half)], osems[s2])
        ow1 = pltpu.async_copy(
            stgs[s2].at[pl.ds(half, half)], out_hbm.at[pl.ds(off1, half)], osems[s2])
        owaits[s2] = (ow0, ow1)

    for s in range(2):
        if owaits[s] is not None:
            for ow in owaits[s]:
                ow.wait()


def kernel(input, weight):
    wt = weight.T
    it = input.T
    tail = lax.slice(weight, (NEMB - TAIL, 0), (NEMB, DIM))
    tbl_flat, idx_lin = _stage1(wt, it, tail)
    table = tbl_flat.reshape(NEMB, DIM)
    out_flat = _stage2(table, idx_lin)
    out5 = out_flat.reshape(NF, 2, BATCH // 128, 8, 128)
    return out5.transpose(2, 4, 0, 1, 3).reshape(BATCH, NF, DIM)
